# Initial kernel scaffold; baseline (speedup 1.0000x reference)
import jax
import jax.numpy as jnp
from jax.experimental import pallas as pl


def kernel(x0, x1, edge_index, edge_type, proj0_W, proj0_b, proj1_W, proj1_b, w1, q1, k1, b1, w2, q2, k2, b2, lin_W, lin_b):
    raise NotImplementedError("write your pallas kernel here")


# speedup vs baseline: 17.5733x; 17.5733x over previous
import functools

import jax
import jax.numpy as jnp
from jax import lax
from jax.experimental import pallas as pl
from jax.experimental.pallas import tpu as pltpu
from jax.experimental.pallas import tpu_sc as plsc

N = 50000
N0 = 25000
E = 800000
R = 4
G = 128
NGROUPS = 6400
E_PAD = NGROUPS * G
NC = 2
NS = 16
NW = NC * NS

K1_WG = NGROUPS // NW
K1_CG = 8
K1_NCH = K1_WG // K1_CG
K1_CE = K1_CG * G

K2_TG = NGROUPS // NS
K2_CG = 4
K2_NCH = K2_TG // K2_CG
K2_CE = K2_CG * G

DEN_ROWS = 3136
DEN_ROWS_LAST = N - 15 * DEN_ROWS


def _iota16():
    return lax.iota(jnp.int32, 16)


def _zero_w8(ref, rows):
    z = jnp.zeros((16,), jnp.float32)

    def body(t, carry):
        lin = t * 16 + _iota16()
        plsc.store_scatter(ref, [lin // 8, lin % 8], z)
        return carry
    lax.fori_loop(0, rows // 2, body, 0)


def _make_k1(H):
    mesh = plsc.VectorSubcoreMesh(core_axis_name="c", subcore_axis_name="s",
                                  num_cores=NC, num_subcores=NS)
    out_type = (
        jax.ShapeDtypeStruct((H, E_PAD), jnp.float32),
        jax.ShapeDtypeStruct((2, N, 8), jnp.float32),
    )
    scratch = [
        pltpu.VMEM((K1_CG, G), jnp.int32),
        pltpu.VMEM((K1_CG, G), jnp.int32),
        pltpu.VMEM((K1_CG, G), jnp.int32),
        pltpu.VMEM((K1_CG, G), jnp.int32),
        pltpu.VMEM((K1_CG, G), jnp.int32),
        pltpu.VMEM((K1_CE, 16), jnp.float32),
        pltpu.VMEM((K1_CE, 16), jnp.float32),
        pltpu.VMEM((K1_CE, 8), jnp.float32),
        pltpu.VMEM((H, K1_CE), jnp.float32),
        pltpu.VMEM((DEN_ROWS, 8), jnp.float32),
        pltpu.VMEM_SHARED((N, 8), jnp.float32),
        pltpu.SemaphoreType.DMA,
        pltpu.SemaphoreType.DMA,
    ]

    @functools.partial(pl.kernel, out_type=out_type, mesh=mesh,
                       compiler_params=pltpu.CompilerParams(
                           use_tc_tiling_on_sc=False, needs_layout_passes=False),
                       scratch_types=scratch)
    def k1(qk_hbm, srcG, dstG, etG, ex_out, den_out,
           src_b, dst_b, et_b, rowsD, rowsS, qkd, qks, pair, exb,
           zbuf, den_sp, sem0, sem1):
        c = lax.axis_index("c")
        s = lax.axis_index("s")
        wid = s * NC + c

        _zero_w8(pair, K1_CE)
        _zero_w8(zbuf, DEN_ROWS)

        @pl.when(s < 15)
        def _():
            pltpu.sync_copy(zbuf, den_sp.at[pl.ds(s * DEN_ROWS, DEN_ROWS)])

        @pl.when(s == 15)
        def _():
            pltpu.sync_copy(zbuf.at[pl.ds(0, DEN_ROWS_LAST)],
                            den_sp.at[pl.ds(15 * DEN_ROWS, DEN_ROWS_LAST)])

        plsc.subcore_barrier()

        def chunk(ci, _):
            base_g = wid * K1_WG + ci * K1_CG
            pltpu.sync_copy(srcG.at[pl.ds(base_g, K1_CG)], src_b)
            pltpu.sync_copy(dstG.at[pl.ds(base_g, K1_CG)], dst_b)
            pltpu.sync_copy(etG.at[pl.ds(base_g, K1_CG)], et_b)

            def rows_body(t, _2):
                g = t // 8
                k = (t % 8) * 16
                sv = src_b[g, pl.ds(k, 16)]
                dv = dst_b[g, pl.ds(k, 16)]
                ev = et_b[g, pl.ds(k, 16)]
                evN = ev * N
                rowsD[g, pl.ds(k, 16)] = evN + dv
                rowsS[g, pl.ds(k, 16)] = evN + sv
                return _2
            lax.fori_loop(0, K1_CG * 8, rows_body, 0)

            cps = []
            for j in range(K1_CG):
                cps.append(pltpu.async_copy(
                    qk_hbm.at[rowsD.at[j]], qkd.at[pl.ds(j * G, G)], sem0))
                cps.append(pltpu.async_copy(
                    qk_hbm.at[rowsS.at[j]], qks.at[pl.ds(j * G, G)], sem1))
            for cp in cps:
                cp.wait()

            edge0 = (wid * K1_WG + ci * K1_CG) * G

            def ex_body(t, _2):
                rvec = t * 16 + _iota16()
                eid = edge0 + rvec
                valid = eid < E
                for h in range(H):
                    qh = plsc.load_gather(qkd, [rvec, jnp.full((16,), h, jnp.int32)])
                    kh = plsc.load_gather(qks, [rvec, jnp.full((16,), H + h, jnp.int32)])
                    sc = qh + kh
                    a = jnp.where(sc >= 0.0, sc, 0.2 * sc)
                    exv = jnp.where(valid, jnp.exp(a), 0.0)
                    exb[h, pl.ds(t * 16, 16)] = exv
                    plsc.store_scatter(pair, [rvec, jnp.full((16,), h, jnp.int32)], exv)
                return _2
            lax.fori_loop(0, K1_CG * 8, ex_body, 0)

            for h in range(H):
                pltpu.sync_copy(exb.at[h], ex_out.at[h, pl.ds(edge0, K1_CE)])

            cps = []
            for j in range(K1_CG):
                cps.append(pltpu.async_copy(
                    pair.at[pl.ds(j * G, G)], den_sp.at[dst_b.at[j]], sem0,
                    add=True))
            for cp in cps:
                cp.wait()
            return _
        lax.fori_loop(0, K1_NCH, chunk, 0)

        plsc.subcore_barrier()

        @pl.when(s < 15)
        def _():
            pltpu.sync_copy(den_sp.at[pl.ds(s * DEN_ROWS, DEN_ROWS)],
                            den_out.at[c, pl.ds(s * DEN_ROWS, DEN_ROWS)])

        @pl.when(s == 15)
        def _():
            pltpu.sync_copy(den_sp.at[pl.ds(15 * DEN_ROWS, DEN_ROWS_LAST)],
                            den_out.at[c, pl.ds(15 * DEN_ROWS, DEN_ROWS_LAST)])

    return k1


def _make_k2(W, h_from_core, H):
    mesh = plsc.VectorSubcoreMesh(core_axis_name="c", subcore_axis_name="s",
                                  num_cores=NC, num_subcores=NS)
    out_type = jax.ShapeDtypeStruct((2, N, W), jnp.float32)
    scratch = [
        pltpu.VMEM((K2_CG, G), jnp.int32),
        pltpu.VMEM((K2_CG, G), jnp.int32),
        pltpu.VMEM((K2_CG, G), jnp.int32),
        pltpu.VMEM((K2_CG, G), jnp.int32),
        pltpu.VMEM((K2_CE, W), jnp.float32),
        pltpu.VMEM((K2_CE,), jnp.float32),
        pltpu.VMEM_SHARED((N, W), jnp.float32),
        pltpu.SemaphoreType.DMA,
    ]
    NV = W // 16

    @functools.partial(pl.kernel, out_type=out_type, mesh=mesh,
                       compiler_params=pltpu.CompilerParams(
                           use_tc_tiling_on_sc=False, needs_layout_passes=False),
                       scratch_types=scratch)
    def k2(xr_hbm, srcG, dstG, etG, ex_hbm, agg_out,
           src_b, dst_b, et_b, rows, msg, exbuf, agg_sp, sem0):
        c = lax.axis_index("c")
        s = lax.axis_index("s")
        c_off = c * (R * N)

        def zero_m16(t, _):
            r = t // NV
            v = (t % NV) * 16
            msg[r, pl.ds(v, 16)] = jnp.zeros((16,), jnp.float32)
            return _
        lax.fori_loop(0, K2_CE * NV, zero_m16, 0)

        base_row = s * DEN_ROWS

        nfull, rem = divmod(DEN_ROWS, K2_CE)
        nfull_l, rem_l = divmod(DEN_ROWS_LAST, K2_CE)

        @pl.when(s < 15)
        def _():
            for q in range(nfull):
                pltpu.sync_copy(msg, agg_sp.at[pl.ds(base_row + q * K2_CE, K2_CE)])
            if rem:
                pltpu.sync_copy(msg.at[pl.ds(0, rem)],
                                agg_sp.at[pl.ds(base_row + nfull * K2_CE, rem)])

        @pl.when(s == 15)
        def _():
            for q in range(nfull_l):
                pltpu.sync_copy(msg, agg_sp.at[pl.ds(base_row + q * K2_CE, K2_CE)])
            if rem_l:
                pltpu.sync_copy(msg.at[pl.ds(0, rem_l)],
                                agg_sp.at[pl.ds(base_row + nfull_l * K2_CE, rem_l)])

        plsc.subcore_barrier()

        def chunk(ci, _):
            base_g = s * K2_TG + ci * K2_CG
            pltpu.sync_copy(srcG.at[pl.ds(base_g, K2_CG)], src_b)
            pltpu.sync_copy(dstG.at[pl.ds(base_g, K2_CG)], dst_b)
            pltpu.sync_copy(etG.at[pl.ds(base_g, K2_CG)], et_b)

            def rows_body(t, _2):
                g = t // 8
                k = (t % 8) * 16
                sv = src_b[g, pl.ds(k, 16)]
                ev = et_b[g, pl.ds(k, 16)]
                rows[g, pl.ds(k, 16)] = c_off + ev * N + sv
                return _2
            lax.fori_loop(0, K2_CG * 8, rows_body, 0)

            edge0 = base_g * G
            if h_from_core:
                pltpu.sync_copy(ex_hbm.at[c, pl.ds(edge0, K2_CE)], exbuf)
            else:
                pltpu.sync_copy(ex_hbm.at[0, pl.ds(edge0, K2_CE)], exbuf)

            cps = []
            for j in range(K2_CG):
                cps.append(pltpu.async_copy(
                    xr_hbm.at[rows.at[j]], msg.at[pl.ds(j * G, G)], sem0))
            for cp in cps:
                cp.wait()

            def sc_body(t, _2):
                for u in range(4):
                    e = t * 4 + u
                    sv = plsc.load_gather(exbuf, [jnp.full((16,), e, jnp.int32)])
                    for v in range(NV):
                        blk = msg[e, pl.ds(v * 16, 16)]
                        msg[e, pl.ds(v * 16, 16)] = blk * sv
                return _2
            lax.fori_loop(0, K2_CE // 4, sc_body, 0)

            cps = []
            for j in range(K2_CG):
                cps.append(pltpu.async_copy(
                    msg.at[pl.ds(j * G, G)], agg_sp.at[dst_b.at[j]], sem0,
                    add=True))
            for cp in cps:
                cp.wait()
            return _
        lax.fori_loop(0, K2_NCH, chunk, 0)

        plsc.subcore_barrier()

        @pl.when(s < 15)
        def _():
            pltpu.sync_copy(agg_sp.at[pl.ds(base_row, DEN_ROWS)],
                            agg_out.at[c, pl.ds(base_row, DEN_ROWS)])

        @pl.when(s == 15)
        def _():
            pltpu.sync_copy(agg_sp.at[pl.ds(base_row, DEN_ROWS_LAST)],
                            agg_out.at[c, pl.ds(base_row, DEN_ROWS_LAST)])

    return k2


def _tc_proj(xs, Ws, bs):
    NB = 1000

    def body(x_ref, w_ref, b_ref, o_ref):
        m = jnp.dot(x_ref[0], w_ref[0], preferred_element_type=jnp.float32)
        o_ref[0] = jnp.maximum(m + b_ref[0, 0], 0.0)

    return pl.pallas_call(
        body,
        grid=(2, N0 // NB),
        in_specs=[
            pl.BlockSpec((1, NB, 256), lambda t, nb: (t, nb, 0)),
            pl.BlockSpec((1, 256, 64), lambda t, nb: (t, 0, 0)),
            pl.BlockSpec((1, 1, 64), lambda t, nb: (t, 0, 0)),
        ],
        out_specs=pl.BlockSpec((1, NB, 64), lambda t, nb: (t, nb, 0)),
        out_shape=jax.ShapeDtypeStruct((2, N0, 64), jnp.float32),
    )(xs, Ws, bs)


def _tc_tables(x, w, qkw, W):
    NB = 2000
    F = x.shape[1]

    def body(x_ref, w_ref, qkw_ref, xr_ref, qk_ref):
        m = jnp.dot(x_ref[...], w_ref[0], preferred_element_type=jnp.float32)
        xr_ref[0, 0] = m[:, :W]
        xr_ref[1, 0] = m[:, W:]
        qk_ref[0] = jnp.dot(x_ref[...], qkw_ref[0],
                            preferred_element_type=jnp.float32)

    return pl.pallas_call(
        body,
        grid=(R, N // NB),
        in_specs=[
            pl.BlockSpec((NB, F), lambda r, nb: (nb, 0)),
            pl.BlockSpec((1, F, 2 * W), lambda r, nb: (r, 0, 0)),
            pl.BlockSpec((1, F, 16), lambda r, nb: (r, 0, 0)),
        ],
        out_specs=[
            pl.BlockSpec((2, 1, NB, W), lambda r, nb: (0, r, nb, 0)),
            pl.BlockSpec((1, NB, 16), lambda r, nb: (r, nb, 0)),
        ],
        out_shape=[
            jax.ShapeDtypeStruct((2, R, N, W), jnp.float32),
            jax.ShapeDtypeStruct((R, N, 16), jnp.float32),
        ],
    )(x, w, qkw)


def _tc_layer1_epilogue(agg, den, b1):
    NB = 2000

    def body(a_ref, d_ref, b_ref, o_ref):
        dsum = d_ref[0] + d_ref[1]
        r0 = 1.0 / (dsum[:, 0:1] + 1e-16)
        r1 = 1.0 / (dsum[:, 1:2] + 1e-16)
        y = jnp.concatenate([a_ref[0] * r0, a_ref[1] * r1], axis=1)
        o_ref[...] = jnp.maximum(y + b_ref[...], 0.0)

    return pl.pallas_call(
        body,
        grid=(N // NB,),
        in_specs=[
            pl.BlockSpec((2, NB, 32), lambda nb: (0, nb, 0)),
            pl.BlockSpec((2, NB, 8), lambda nb: (0, nb, 0)),
            pl.BlockSpec((64,), lambda nb: (0,)),
        ],
        out_specs=pl.BlockSpec((NB, 64), lambda nb: (nb, 0)),
        out_shape=jax.ShapeDtypeStruct((N, 64), jnp.float32),
    )(agg, den, b1)


def _tc_head(agg2, den2, b2, lin_W, lin_b):
    NB = 1000

    def body(a_ref, d_ref, b_ref, lw_ref, lb_ref, o_ref):
        dsum = d_ref[0] + d_ref[1]
        r0 = 1.0 / (dsum[:, 0:1] + 1e-16)
        y = jnp.concatenate([a_ref[0], a_ref[1]], axis=1) * r0
        y = jnp.maximum(y + b_ref[...], 0.0)
        o_ref[...] = jnp.dot(y, lw_ref[...],
                             preferred_element_type=jnp.float32) + lb_ref[...]

    return pl.pallas_call(
        body,
        grid=(N0 // NB,),
        in_specs=[
            pl.BlockSpec((2, NB, 16), lambda nb: (0, nb, 0)),
            pl.BlockSpec((2, NB, 8), lambda nb: (0, nb, 0)),
            pl.BlockSpec((32,), lambda nb: (0,)),
            pl.BlockSpec((32, 8), lambda nb: (0, 0)),
            pl.BlockSpec((8,), lambda nb: (0,)),
        ],
        out_specs=pl.BlockSpec((NB, 8), lambda nb: (nb, 0)),
        out_shape=jax.ShapeDtypeStruct((N0, 8), jnp.float32),
    )(agg2, den2, b2, lin_W, lin_b)


_SC_KERNELS = {}


def _sc_kernels():
    if not _SC_KERNELS:
        _SC_KERNELS["k1l1"] = _make_k1(2)
        _SC_KERNELS["k1l2"] = _make_k1(1)
        _SC_KERNELS["k2l1"] = _make_k2(32, True, 2)
        _SC_KERNELS["k2l2"] = _make_k2(16, False, 1)
    return _SC_KERNELS


def _qkw(w, q, k, H):
    qk = jnp.einsum('rio,oh->rih', w, jnp.concatenate([q, k], axis=1))
    pad = 16 - 2 * H
    return jnp.pad(qk, ((0, 0), (0, 0), (0, pad)))


def kernel(x0, x1, edge_index, edge_type, proj0_W, proj0_b, proj1_W, proj1_b,
           w1, q1, k1, b1, w2, q2, k2, b2, lin_W, lin_b):
    ei = edge_index.astype(jnp.int32)
    et = edge_type.astype(jnp.int32)
    padn = E_PAD - E
    srcG = jnp.pad(ei[0], (0, padn)).reshape(NGROUPS, G)
    dstG = jnp.pad(ei[1], (0, padn)).reshape(NGROUPS, G)
    etG = jnp.pad(et, (0, padn)).reshape(NGROUPS, G)

    xs = jnp.stack([x0, x1])
    Ws = jnp.stack([proj0_W, proj1_W])
    bs = jnp.stack([proj0_b, proj1_b]).reshape(2, 1, 64)

    x = _tc_proj(xs, Ws, bs).reshape(N, 64)

    xr1, qk1 = _tc_tables(x, w1, _qkw(w1, q1, k1, 2), 32)
    xr1 = xr1.reshape(2 * R * N, 32)
    qk1 = qk1.reshape(R * N, 16)
    sck = _sc_kernels()
    ex1, den1 = sck["k1l1"](qk1, srcG, dstG, etG)
    agg1 = sck["k2l1"](xr1, srcG, dstG, etG, ex1)
    x2 = _tc_layer1_epilogue(agg1, den1, b1)

    xr2, qk2 = _tc_tables(x2, w2, _qkw(w2, q2, k2, 1), 16)
    xr2 = xr2.reshape(2 * R * N, 16)
    qk2 = qk2.reshape(R * N, 16)
    ex2, den2 = sck["k1l2"](qk2, srcG, dstG, etG)
    agg2 = sck["k2l2"](xr2, srcG, dstG, etG, ex2)

    return _tc_head(agg2, den2, b2, lin_W, lin_b)


# speedup vs baseline: 17.8500x; 1.0157x over previous
import functools

import jax
import jax.numpy as jnp
from jax import lax
from jax.experimental import pallas as pl
from jax.experimental.pallas import tpu as pltpu
from jax.experimental.pallas import tpu_sc as plsc

N = 50000
N0 = 25000
E = 800000
R = 4
G = 128
NGROUPS = 6400
E_PAD = NGROUPS * G
NC = 2
NS = 16
NW = NC * NS

K1_WG = NGROUPS // NW
K1_CG = 8
K1_NCH = K1_WG // K1_CG
K1_CE = K1_CG * G

K2_TG = NGROUPS // NS
K2_CG = 4
K2_NCH = K2_TG // K2_CG
K2_CE = K2_CG * G

DEN_ROWS = 3136
DEN_ROWS_LAST = N - 15 * DEN_ROWS


def _iota16():
    return lax.iota(jnp.int32, 16)


def _zero_wk(ref, rows, width):
    z = jnp.zeros((16,), jnp.float32)

    def body(t, carry):
        lin = t * 16 + _iota16()
        plsc.store_scatter(ref, [lin // width, lin % width], z)
        return carry
    lax.fori_loop(0, rows * width // 16, body, 0)


def _zero_w8(ref, rows):
    _zero_wk(ref, rows, 8)


def _make_k1(H):
    mesh = plsc.VectorSubcoreMesh(core_axis_name="c", subcore_axis_name="s",
                                  num_cores=NC, num_subcores=NS)
    out_type = (
        jax.ShapeDtypeStruct((H, E_PAD), jnp.float32),
        jax.ShapeDtypeStruct((2, N, 8), jnp.float32),
    )
    scratch = [
        pltpu.VMEM((K1_CG, G), jnp.int32),
        pltpu.VMEM((K1_CG, G), jnp.int32),
        pltpu.VMEM((K1_CG, G), jnp.int32),
        pltpu.VMEM((K1_CG, G), jnp.int32),
        pltpu.VMEM((K1_CG, G), jnp.int32),
        pltpu.VMEM((K1_CE, 16), jnp.float32),
        pltpu.VMEM((K1_CE, 16), jnp.float32),
        pltpu.VMEM((K1_CE, 8), jnp.float32),
        pltpu.VMEM((H, K1_CE), jnp.float32),
        pltpu.VMEM((DEN_ROWS, 8), jnp.float32),
        pltpu.VMEM_SHARED((N, 8), jnp.float32),
        pltpu.SemaphoreType.DMA,
        pltpu.SemaphoreType.DMA,
    ]

    @functools.partial(pl.kernel, out_type=out_type, mesh=mesh,
                       compiler_params=pltpu.CompilerParams(
                           use_tc_tiling_on_sc=False, needs_layout_passes=False),
                       scratch_types=scratch)
    def k1(qk_hbm, srcG, dstG, etG, ex_out, den_out,
           src_b, dst_b, et_b, rowsD, rowsS, qkd, qks, pair, exb,
           zbuf, den_sp, sem0, sem1):
        c = lax.axis_index("c")
        s = lax.axis_index("s")
        wid = s * NC + c

        _zero_w8(pair, K1_CE)
        _zero_w8(zbuf, DEN_ROWS)

        @pl.when(s < 15)
        def _():
            pltpu.sync_copy(zbuf, den_sp.at[pl.ds(s * DEN_ROWS, DEN_ROWS)])

        @pl.when(s == 15)
        def _():
            pltpu.sync_copy(zbuf.at[pl.ds(0, DEN_ROWS_LAST)],
                            den_sp.at[pl.ds(15 * DEN_ROWS, DEN_ROWS_LAST)])

        plsc.subcore_barrier()

        def chunk(ci, _):
            base_g = wid * K1_WG + ci * K1_CG
            pltpu.sync_copy(srcG.at[pl.ds(base_g, K1_CG)], src_b)
            pltpu.sync_copy(dstG.at[pl.ds(base_g, K1_CG)], dst_b)
            pltpu.sync_copy(etG.at[pl.ds(base_g, K1_CG)], et_b)

            def rows_body(t, _2):
                g = t // 8
                k = (t % 8) * 16
                sv = src_b[g, pl.ds(k, 16)]
                dv = dst_b[g, pl.ds(k, 16)]
                ev = et_b[g, pl.ds(k, 16)]
                evN = ev * N
                rowsD[g, pl.ds(k, 16)] = evN + dv
                rowsS[g, pl.ds(k, 16)] = evN + sv
                return _2
            lax.fori_loop(0, K1_CG * 8, rows_body, 0)

            cps = []
            for j in range(K1_CG):
                cps.append(pltpu.async_copy(
                    qk_hbm.at[rowsD.at[j]], qkd.at[pl.ds(j * G, G)], sem0))
                cps.append(pltpu.async_copy(
                    qk_hbm.at[rowsS.at[j]], qks.at[pl.ds(j * G, G)], sem1))
            for cp in cps:
                cp.wait()

            edge0 = (wid * K1_WG + ci * K1_CG) * G

            def ex_body(t, _2):
                rvec = t * 16 + _iota16()
                eid = edge0 + rvec
                valid = eid < E
                for h in range(H):
                    qh = plsc.load_gather(qkd, [rvec, jnp.full((16,), h, jnp.int32)])
                    kh = plsc.load_gather(qks, [rvec, jnp.full((16,), H + h, jnp.int32)])
                    sc = qh + kh
                    a = jnp.where(sc >= 0.0, sc, 0.2 * sc)
                    exv = jnp.where(valid, jnp.exp(a), 0.0)
                    exb[h, pl.ds(t * 16, 16)] = exv
                    plsc.store_scatter(pair, [rvec, jnp.full((16,), h, jnp.int32)], exv)
                return _2
            lax.fori_loop(0, K1_CG * 8, ex_body, 0)

            for h in range(H):
                pltpu.sync_copy(exb.at[h], ex_out.at[h, pl.ds(edge0, K1_CE)])

            cps = []
            for j in range(K1_CG):
                cps.append(pltpu.async_copy(
                    pair.at[pl.ds(j * G, G)], den_sp.at[dst_b.at[j]], sem0,
                    add=True))
            for cp in cps:
                cp.wait()
            return _
        lax.fori_loop(0, K1_NCH, chunk, 0)

        plsc.subcore_barrier()

        @pl.when(s < 15)
        def _():
            pltpu.sync_copy(den_sp.at[pl.ds(s * DEN_ROWS, DEN_ROWS)],
                            den_out.at[c, pl.ds(s * DEN_ROWS, DEN_ROWS)])

        @pl.when(s == 15)
        def _():
            pltpu.sync_copy(den_sp.at[pl.ds(15 * DEN_ROWS, DEN_ROWS_LAST)],
                            den_out.at[c, pl.ds(15 * DEN_ROWS, DEN_ROWS_LAST)])

    return k1


def _make_k2(W, h_from_core, H):
    mesh = plsc.VectorSubcoreMesh(core_axis_name="c", subcore_axis_name="s",
                                  num_cores=NC, num_subcores=NS)
    out_type = jax.ShapeDtypeStruct((2, N, W), jnp.float32)
    scratch = [
        pltpu.VMEM((K2_CG, G), jnp.int32),
        pltpu.VMEM((K2_CG, G), jnp.int32),
        pltpu.VMEM((K2_CG, G), jnp.int32),
        pltpu.VMEM((K2_CG, G), jnp.int32),
        pltpu.VMEM((K2_CE, W), jnp.float32),
        pltpu.VMEM((K2_CE,), jnp.float32),
        pltpu.VMEM_SHARED((N, W), jnp.float32),
        pltpu.SemaphoreType.DMA,
    ]
    NV = W // 16

    @functools.partial(pl.kernel, out_type=out_type, mesh=mesh,
                       compiler_params=pltpu.CompilerParams(
                           use_tc_tiling_on_sc=False, needs_layout_passes=False),
                       scratch_types=scratch)
    def k2(xr_hbm, srcG, dstG, etG, ex_hbm, agg_out,
           src_b, dst_b, et_b, rows, msg, exbuf, agg_sp, sem0):
        c = lax.axis_index("c")
        s = lax.axis_index("s")
        c_off = c * (R * N)

        def zero_m16(t, _):
            r = t // NV
            v = (t % NV) * 16
            msg[r, pl.ds(v, 16)] = jnp.zeros((16,), jnp.float32)
            return _
        lax.fori_loop(0, K2_CE * NV, zero_m16, 0)

        base_row = s * DEN_ROWS

        nfull, rem = divmod(DEN_ROWS, K2_CE)
        nfull_l, rem_l = divmod(DEN_ROWS_LAST, K2_CE)

        @pl.when(s < 15)
        def _():
            for q in range(nfull):
                pltpu.sync_copy(msg, agg_sp.at[pl.ds(base_row + q * K2_CE, K2_CE)])
            if rem:
                pltpu.sync_copy(msg.at[pl.ds(0, rem)],
                                agg_sp.at[pl.ds(base_row + nfull * K2_CE, rem)])

        @pl.when(s == 15)
        def _():
            for q in range(nfull_l):
                pltpu.sync_copy(msg, agg_sp.at[pl.ds(base_row + q * K2_CE, K2_CE)])
            if rem_l:
                pltpu.sync_copy(msg.at[pl.ds(0, rem_l)],
                                agg_sp.at[pl.ds(base_row + nfull_l * K2_CE, rem_l)])

        plsc.subcore_barrier()

        def chunk(ci, _):
            base_g = s * K2_TG + ci * K2_CG
            pltpu.sync_copy(srcG.at[pl.ds(base_g, K2_CG)], src_b)
            pltpu.sync_copy(dstG.at[pl.ds(base_g, K2_CG)], dst_b)
            pltpu.sync_copy(etG.at[pl.ds(base_g, K2_CG)], et_b)

            def rows_body(t, _2):
                g = t // 8
                k = (t % 8) * 16
                sv = src_b[g, pl.ds(k, 16)]
                ev = et_b[g, pl.ds(k, 16)]
                rows[g, pl.ds(k, 16)] = c_off + ev * N + sv
                return _2
            lax.fori_loop(0, K2_CG * 8, rows_body, 0)

            edge0 = base_g * G
            if h_from_core:
                pltpu.sync_copy(ex_hbm.at[c, pl.ds(edge0, K2_CE)], exbuf)
            else:
                pltpu.sync_copy(ex_hbm.at[0, pl.ds(edge0, K2_CE)], exbuf)

            cps = []
            for j in range(K2_CG):
                cps.append(pltpu.async_copy(
                    xr_hbm.at[rows.at[j]], msg.at[pl.ds(j * G, G)], sem0))
            for cp in cps:
                cp.wait()

            def sc_body(t, _2):
                for u in range(4):
                    e = t * 4 + u
                    sv = plsc.load_gather(exbuf, [jnp.full((16,), e, jnp.int32)])
                    for v in range(NV):
                        blk = msg[e, pl.ds(v * 16, 16)]
                        msg[e, pl.ds(v * 16, 16)] = blk * sv
                return _2
            lax.fori_loop(0, K2_CE // 4, sc_body, 0)

            cps = []
            for j in range(K2_CG):
                cps.append(pltpu.async_copy(
                    msg.at[pl.ds(j * G, G)], agg_sp.at[dst_b.at[j]], sem0,
                    add=True))
            for cp in cps:
                cp.wait()
            return _
        lax.fori_loop(0, K2_NCH, chunk, 0)

        plsc.subcore_barrier()

        @pl.when(s < 15)
        def _():
            pltpu.sync_copy(agg_sp.at[pl.ds(base_row, DEN_ROWS)],
                            agg_out.at[c, pl.ds(base_row, DEN_ROWS)])

        @pl.when(s == 15)
        def _():
            pltpu.sync_copy(agg_sp.at[pl.ds(base_row, DEN_ROWS_LAST)],
                            agg_out.at[c, pl.ds(base_row, DEN_ROWS_LAST)])

    return k2


KM_CG = 4
KM_CE = KM_CG * G
KM_NCH = K2_TG // KM_CG


def _make_km2():
    W = 16
    mesh = plsc.VectorSubcoreMesh(core_axis_name="c", subcore_axis_name="s",
                                  num_cores=NC, num_subcores=NS)
    out_type = (
        jax.ShapeDtypeStruct((2, N, W), jnp.float32),
        jax.ShapeDtypeStruct((2, N, 8), jnp.float32),
    )
    scratch = [
        pltpu.VMEM((KM_CG, G), jnp.int32),
        pltpu.VMEM((KM_CG, G), jnp.int32),
        pltpu.VMEM((KM_CG, G), jnp.int32),
        pltpu.VMEM((KM_CG, G), jnp.int32),
        pltpu.VMEM((KM_CG, G), jnp.int32),
        pltpu.VMEM((KM_CG, G), jnp.int32),
        pltpu.VMEM((KM_CE, 16), jnp.float32),
        pltpu.VMEM((KM_CE, 16), jnp.float32),
        pltpu.VMEM((KM_CE, W), jnp.float32),
        pltpu.VMEM((KM_CE, 8), jnp.float32),
        pltpu.VMEM((KM_CE,), jnp.float32),
        pltpu.VMEM_SHARED((N, W), jnp.float32),
        pltpu.VMEM_SHARED((N, 8), jnp.float32),
        pltpu.SemaphoreType.DMA,
        pltpu.SemaphoreType.DMA,
    ]

    @functools.partial(pl.kernel, out_type=out_type, mesh=mesh,
                       compiler_params=pltpu.CompilerParams(
                           use_tc_tiling_on_sc=False, needs_layout_passes=False),
                       scratch_types=scratch)
    def km2(xr_hbm, qk_hbm, srcG, dstG, etG, agg_out, den_out,
            src_b, dst_b, et_b, rowsD, rowsS, rowsM, qkd, qks, msg, pair,
            exbuf, agg_sp, den_sp, sem0, sem1):
        c = lax.axis_index("c")
        s = lax.axis_index("s")
        c_off = c * (R * N)

        def zero_m16(t, _):
            msg[t, pl.ds(0, 16)] = jnp.zeros((16,), jnp.float32)
            return _
        lax.fori_loop(0, KM_CE, zero_m16, 0)
        _zero_wk(pair, KM_CE, 8)

        base_row = s * DEN_ROWS

        @pl.when(s < 15)
        def _():
            for q in range(DEN_ROWS // KM_CE):
                pltpu.sync_copy(msg, agg_sp.at[pl.ds(base_row + q * KM_CE, KM_CE)])
                pltpu.sync_copy(pair, den_sp.at[pl.ds(base_row + q * KM_CE, KM_CE)])
            rem = DEN_ROWS % KM_CE
            if rem:
                off = base_row + (DEN_ROWS // KM_CE) * KM_CE
                pltpu.sync_copy(msg.at[pl.ds(0, rem)], agg_sp.at[pl.ds(off, rem)])
                pltpu.sync_copy(pair.at[pl.ds(0, rem)], den_sp.at[pl.ds(off, rem)])

        @pl.when(s == 15)
        def _():
            for q in range(DEN_ROWS_LAST // KM_CE):
                pltpu.sync_copy(msg, agg_sp.at[pl.ds(base_row + q * KM_CE, KM_CE)])
                pltpu.sync_copy(pair, den_sp.at[pl.ds(base_row + q * KM_CE, KM_CE)])
            rem = DEN_ROWS_LAST % KM_CE
            if rem:
                off = base_row + (DEN_ROWS_LAST // KM_CE) * KM_CE
                pltpu.sync_copy(msg.at[pl.ds(0, rem)], agg_sp.at[pl.ds(off, rem)])
                pltpu.sync_copy(pair.at[pl.ds(0, rem)], den_sp.at[pl.ds(off, rem)])

        plsc.subcore_barrier()

        def chunk(ci, _):
            base_g = s * K2_TG + ci * KM_CG
            pltpu.sync_copy(srcG.at[pl.ds(base_g, KM_CG)], src_b)
            pltpu.sync_copy(dstG.at[pl.ds(base_g, KM_CG)], dst_b)
            pltpu.sync_copy(etG.at[pl.ds(base_g, KM_CG)], et_b)

            def rows_body(t, _2):
                g = t // 8
                k = (t % 8) * 16
                sv = src_b[g, pl.ds(k, 16)]
                dv = dst_b[g, pl.ds(k, 16)]
                ev = et_b[g, pl.ds(k, 16)]
                evN = ev * N
                rowsD[g, pl.ds(k, 16)] = evN + dv
                rs = evN + sv
                rowsS[g, pl.ds(k, 16)] = rs
                rowsM[g, pl.ds(k, 16)] = rs + c_off
                return _2
            lax.fori_loop(0, KM_CG * 8, rows_body, 0)

            cps = []
            for j in range(KM_CG):
                cps.append(pltpu.async_copy(
                    qk_hbm.at[rowsD.at[j]], qkd.at[pl.ds(j * G, G)], sem0))
                cps.append(pltpu.async_copy(
                    qk_hbm.at[rowsS.at[j]], qks.at[pl.ds(j * G, G)], sem0))
                cps.append(pltpu.async_copy(
                    xr_hbm.at[rowsM.at[j]], msg.at[pl.ds(j * G, G)], sem1))
            for cp in cps:
                cp.wait()

            edge0 = base_g * G

            def ex_body(t, _2):
                rvec = t * 16 + _iota16()
                eid = edge0 + rvec
                valid = eid < E
                q = plsc.load_gather(qkd, [rvec, jnp.full((16,), 0, jnp.int32)])
                k = plsc.load_gather(qks, [rvec, jnp.full((16,), 1, jnp.int32)])
                sc = q + k
                a = jnp.where(sc >= 0.0, sc, 0.2 * sc)
                exv = jnp.where(valid, jnp.exp(a), 0.0)
                exbuf[pl.ds(t * 16, 16)] = exv
                plsc.store_scatter(pair, [rvec, jnp.full((16,), 0, jnp.int32)], exv)
                return _2
            lax.fori_loop(0, KM_CE // 16, ex_body, 0)

            def sc_body(t, _2):
                for u in range(4):
                    e = t * 4 + u
                    sv = plsc.load_gather(exbuf, [jnp.full((16,), e, jnp.int32)])
                    msg[e, pl.ds(0, 16)] = msg[e, pl.ds(0, 16)] * sv
                return _2
            lax.fori_loop(0, KM_CE // 4, sc_body, 0)

            cps = []
            for j in range(KM_CG):
                cps.append(pltpu.async_copy(
                    msg.at[pl.ds(j * G, G)], agg_sp.at[dst_b.at[j]], sem0,
                    add=True))
                cps.append(pltpu.async_copy(
                    pair.at[pl.ds(j * G, G)], den_sp.at[dst_b.at[j]], sem1,
                    add=True))
            for cp in cps:
                cp.wait()
            return _
        lax.fori_loop(0, KM_NCH, chunk, 0)

        plsc.subcore_barrier()

        @pl.when(s < 15)
        def _():
            pltpu.sync_copy(agg_sp.at[pl.ds(base_row, DEN_ROWS)],
                            agg_out.at[c, pl.ds(base_row, DEN_ROWS)])
            pltpu.sync_copy(den_sp.at[pl.ds(base_row, DEN_ROWS)],
                            den_out.at[c, pl.ds(base_row, DEN_ROWS)])

        @pl.when(s == 15)
        def _():
            pltpu.sync_copy(agg_sp.at[pl.ds(base_row, DEN_ROWS_LAST)],
                            agg_out.at[c, pl.ds(base_row, DEN_ROWS_LAST)])
            pltpu.sync_copy(den_sp.at[pl.ds(base_row, DEN_ROWS_LAST)],
                            den_out.at[c, pl.ds(base_row, DEN_ROWS_LAST)])

    return km2


def _tc_proj(xs, Ws, bs):
    NB = 1000

    def body(x_ref, w_ref, b_ref, o_ref):
        m = jnp.dot(x_ref[0], w_ref[0], preferred_element_type=jnp.float32)
        o_ref[0] = jnp.maximum(m + b_ref[0, 0], 0.0)

    return pl.pallas_call(
        body,
        grid=(2, N0 // NB),
        in_specs=[
            pl.BlockSpec((1, NB, 256), lambda t, nb: (t, nb, 0)),
            pl.BlockSpec((1, 256, 64), lambda t, nb: (t, 0, 0)),
            pl.BlockSpec((1, 1, 64), lambda t, nb: (t, 0, 0)),
        ],
        out_specs=pl.BlockSpec((1, NB, 64), lambda t, nb: (t, nb, 0)),
        out_shape=jax.ShapeDtypeStruct((2, N0, 64), jnp.float32),
    )(xs, Ws, bs)


def _tc_tables(x, w, qkw, W):
    NB = 2000
    F = x.shape[1]

    def body(x_ref, w_ref, qkw_ref, xr_ref, qk_ref):
        m = jnp.dot(x_ref[...], w_ref[0], preferred_element_type=jnp.float32)
        xr_ref[0, 0] = m[:, :W]
        xr_ref[1, 0] = m[:, W:]
        qk_ref[0] = jnp.dot(x_ref[...], qkw_ref[0],
                            preferred_element_type=jnp.float32)

    return pl.pallas_call(
        body,
        grid=(R, N // NB),
        in_specs=[
            pl.BlockSpec((NB, F), lambda r, nb: (nb, 0)),
            pl.BlockSpec((1, F, 2 * W), lambda r, nb: (r, 0, 0)),
            pl.BlockSpec((1, F, 16), lambda r, nb: (r, 0, 0)),
        ],
        out_specs=[
            pl.BlockSpec((2, 1, NB, W), lambda r, nb: (0, r, nb, 0)),
            pl.BlockSpec((1, NB, 16), lambda r, nb: (r, nb, 0)),
        ],
        out_shape=[
            jax.ShapeDtypeStruct((2, R, N, W), jnp.float32),
            jax.ShapeDtypeStruct((R, N, 16), jnp.float32),
        ],
    )(x, w, qkw)


def _tc_layer1_epilogue(agg, den, b1):
    NB = 2000

    def body(a_ref, d_ref, b_ref, o_ref):
        dsum = d_ref[0] + d_ref[1]
        r0 = 1.0 / (dsum[:, 0:1] + 1e-16)
        r1 = 1.0 / (dsum[:, 1:2] + 1e-16)
        y = jnp.concatenate([a_ref[0] * r0, a_ref[1] * r1], axis=1)
        o_ref[...] = jnp.maximum(y + b_ref[...], 0.0)

    return pl.pallas_call(
        body,
        grid=(N // NB,),
        in_specs=[
            pl.BlockSpec((2, NB, 32), lambda nb: (0, nb, 0)),
            pl.BlockSpec((2, NB, 8), lambda nb: (0, nb, 0)),
            pl.BlockSpec((64,), lambda nb: (0,)),
        ],
        out_specs=pl.BlockSpec((NB, 64), lambda nb: (nb, 0)),
        out_shape=jax.ShapeDtypeStruct((N, 64), jnp.float32),
    )(agg, den, b1)


def _tc_head(agg2, den2, b2, lin_W, lin_b):
    NB = 1000

    def body(a_ref, d_ref, b_ref, lw_ref, lb_ref, o_ref):
        r0 = 1.0 / (d_ref[0][:, 0:1] + 1e-16)
        y = jnp.concatenate([a_ref[0], a_ref[1]], axis=1) * r0
        y = jnp.maximum(y + b_ref[...], 0.0)
        o_ref[...] = jnp.dot(y, lw_ref[...],
                             preferred_element_type=jnp.float32) + lb_ref[...]

    return pl.pallas_call(
        body,
        grid=(N0 // NB,),
        in_specs=[
            pl.BlockSpec((2, NB, 16), lambda nb: (0, nb, 0)),
            pl.BlockSpec((2, NB, 8), lambda nb: (0, nb, 0)),
            pl.BlockSpec((32,), lambda nb: (0,)),
            pl.BlockSpec((32, 8), lambda nb: (0, 0)),
            pl.BlockSpec((8,), lambda nb: (0,)),
        ],
        out_specs=pl.BlockSpec((NB, 8), lambda nb: (nb, 0)),
        out_shape=jax.ShapeDtypeStruct((N0, 8), jnp.float32),
    )(agg2, den2, b2, lin_W, lin_b)


_SC_KERNELS = {}


def _sc_kernels():
    if not _SC_KERNELS:
        _SC_KERNELS["k1l1"] = _make_k1(2)
        _SC_KERNELS["k2l1"] = _make_k2(32, True, 2)
        _SC_KERNELS["km2"] = _make_km2()
    return _SC_KERNELS


def _qkw(w, q, k, H):
    qk = jnp.einsum('rio,oh->rih', w, jnp.concatenate([q, k], axis=1))
    pad = 16 - 2 * H
    return jnp.pad(qk, ((0, 0), (0, 0), (0, pad)))


def kernel(x0, x1, edge_index, edge_type, proj0_W, proj0_b, proj1_W, proj1_b,
           w1, q1, k1, b1, w2, q2, k2, b2, lin_W, lin_b):
    ei = edge_index.astype(jnp.int32)
    et = edge_type.astype(jnp.int32)
    padn = E_PAD - E
    srcG = jnp.pad(ei[0], (0, padn)).reshape(NGROUPS, G)
    dstG = jnp.pad(ei[1], (0, padn)).reshape(NGROUPS, G)
    etG = jnp.pad(et, (0, padn)).reshape(NGROUPS, G)

    xs = jnp.stack([x0, x1])
    Ws = jnp.stack([proj0_W, proj1_W])
    bs = jnp.stack([proj0_b, proj1_b]).reshape(2, 1, 64)

    x = _tc_proj(xs, Ws, bs).reshape(N, 64)

    xr1, qk1 = _tc_tables(x, w1, _qkw(w1, q1, k1, 2), 32)
    xr1 = xr1.reshape(2 * R * N, 32)
    qk1 = qk1.reshape(R * N, 16)
    sck = _sc_kernels()
    ex1, den1 = sck["k1l1"](qk1, srcG, dstG, etG)
    agg1 = sck["k2l1"](xr1, srcG, dstG, etG, ex1)
    x2 = _tc_layer1_epilogue(agg1, den1, b1)

    xr2, qk2 = _tc_tables(x2, w2, _qkw(w2, q2, k2, 1), 16)
    xr2 = xr2.reshape(2 * R * N, 16)
    qk2 = qk2.reshape(R * N, 16)
    agg2, den2 = sck["km2"](xr2, qk2, srcG, dstG, etG)

    return _tc_head(agg2, den2, b2, lin_W, lin_b)


# speedup vs baseline: 19.0600x; 1.0678x over previous
import functools

import jax
import jax.numpy as jnp
from jax import lax
from jax.experimental import pallas as pl
from jax.experimental.pallas import tpu as pltpu
from jax.experimental.pallas import tpu_sc as plsc

N = 50000
N0 = 25000
E = 800000
R = 4
G = 128
NGROUPS = 6400
E_PAD = NGROUPS * G
NC = 2
NS = 16
NW = NC * NS

K1_WG = NGROUPS // NW
K1_CG = 10
K1_NCH = K1_WG // K1_CG
K1_CE = K1_CG * G

K2_TG = NGROUPS // NS
K2_CG = 5
K2_NCH = K2_TG // K2_CG
K2_CE = K2_CG * G

DEN_ROWS = 3136
DEN_ROWS_LAST = N - 15 * DEN_ROWS


def _iota16():
    return lax.iota(jnp.int32, 16)


def _zero_wk(ref, rows, width):
    z = jnp.zeros((16,), jnp.float32)

    def body(t, carry):
        lin = t * 16 + _iota16()
        plsc.store_scatter(ref, [lin // width, lin % width], z)
        return carry
    lax.fori_loop(0, rows * width // 16, body, 0)


def _zero_w8(ref, rows):
    _zero_wk(ref, rows, 8)


def _make_k1(H):
    mesh = plsc.VectorSubcoreMesh(core_axis_name="c", subcore_axis_name="s",
                                  num_cores=NC, num_subcores=NS)
    out_type = (
        jax.ShapeDtypeStruct((H, E_PAD), jnp.float32),
        jax.ShapeDtypeStruct((2, N, 8), jnp.float32),
    )
    scratch = [
        pltpu.VMEM((K1_CG, 3, G), jnp.int32),
        pltpu.VMEM((K1_CG, G), jnp.int32),
        pltpu.VMEM((K1_CG, G), jnp.int32),
        pltpu.VMEM((K1_CE, 16), jnp.float32),
        pltpu.VMEM((K1_CE, 16), jnp.float32),
        pltpu.VMEM((K1_CE, 8), jnp.float32),
        pltpu.VMEM((H, K1_CE), jnp.float32),
        pltpu.VMEM_SHARED((N, 8), jnp.float32),
        pltpu.SemaphoreType.DMA,
        pltpu.SemaphoreType.DMA,
    ]

    @functools.partial(pl.kernel, out_type=out_type, mesh=mesh,
                       compiler_params=pltpu.CompilerParams(
                           use_tc_tiling_on_sc=False, needs_layout_passes=False),
                       scratch_types=scratch)
    def k1(qk_hbm, eG, ex_out, den_out,
           eb, rowsD, rowsS, qkd, qks, pair, exb,
           den_sp, sem0, sem1):
        c = lax.axis_index("c")
        s = lax.axis_index("s")
        wid = s * NC + c

        _zero_w8(pair, K1_CE)
        nf, rm = divmod(DEN_ROWS, K1_CE)
        nf_l, rm_l = divmod(DEN_ROWS_LAST, K1_CE)
        base_row = s * DEN_ROWS

        @pl.when(s < 15)
        def _():
            for q in range(nf):
                pltpu.sync_copy(pair, den_sp.at[pl.ds(base_row + q * K1_CE, K1_CE)])
            if rm:
                pltpu.sync_copy(pair.at[pl.ds(0, rm)],
                                den_sp.at[pl.ds(base_row + nf * K1_CE, rm)])

        @pl.when(s == 15)
        def _():
            for q in range(nf_l):
                pltpu.sync_copy(pair, den_sp.at[pl.ds(base_row + q * K1_CE, K1_CE)])
            if rm_l:
                pltpu.sync_copy(pair.at[pl.ds(0, rm_l)],
                                den_sp.at[pl.ds(base_row + nf_l * K1_CE, rm_l)])

        plsc.subcore_barrier()

        def chunk(ci, _):
            base_g = wid * K1_WG + ci * K1_CG
            pltpu.sync_copy(eG.at[pl.ds(base_g, K1_CG)], eb)

            def rows_body(t, _2):
                g = t // 8
                k = (t % 8) * 16
                sv = eb[g, 0, pl.ds(k, 16)]
                dv = eb[g, 1, pl.ds(k, 16)]
                ev = eb[g, 2, pl.ds(k, 16)]
                evN = ev * N
                rowsD[g, pl.ds(k, 16)] = evN + dv
                rowsS[g, pl.ds(k, 16)] = evN + sv
                return _2
            lax.fori_loop(0, K1_CG * 8, rows_body, 0)

            cps = []
            for j in range(K1_CG):
                cps.append(pltpu.async_copy(
                    qk_hbm.at[rowsD.at[j]], qkd.at[pl.ds(j * G, G)], sem0))
                cps.append(pltpu.async_copy(
                    qk_hbm.at[rowsS.at[j]], qks.at[pl.ds(j * G, G)], sem1))
            for cp in cps:
                cp.wait()

            edge0 = (wid * K1_WG + ci * K1_CG) * G

            def ex_body(t, _2):
                rvec = t * 16 + _iota16()
                eid = edge0 + rvec
                valid = eid < E
                for h in range(H):
                    qh = plsc.load_gather(qkd, [rvec, jnp.full((16,), h, jnp.int32)])
                    kh = plsc.load_gather(qks, [rvec, jnp.full((16,), H + h, jnp.int32)])
                    sc = qh + kh
                    a = jnp.where(sc >= 0.0, sc, 0.2 * sc)
                    exv = jnp.where(valid, jnp.exp(a), 0.0)
                    exb[h, pl.ds(t * 16, 16)] = exv
                    plsc.store_scatter(pair, [rvec, jnp.full((16,), h, jnp.int32)], exv)
                return _2
            lax.fori_loop(0, K1_CG * 8, ex_body, 0)

            for h in range(H):
                pltpu.sync_copy(exb.at[h], ex_out.at[h, pl.ds(edge0, K1_CE)])

            cps = []
            for j in range(K1_CG):
                cps.append(pltpu.async_copy(
                    pair.at[pl.ds(j * G, G)], den_sp.at[eb.at[j, 1]], sem0,
                    add=True))
            for cp in cps:
                cp.wait()
            return _
        lax.fori_loop(0, K1_NCH, chunk, 0)

        plsc.subcore_barrier()

        @pl.when(s < 15)
        def _():
            pltpu.sync_copy(den_sp.at[pl.ds(s * DEN_ROWS, DEN_ROWS)],
                            den_out.at[c, pl.ds(s * DEN_ROWS, DEN_ROWS)])

        @pl.when(s == 15)
        def _():
            pltpu.sync_copy(den_sp.at[pl.ds(15 * DEN_ROWS, DEN_ROWS_LAST)],
                            den_out.at[c, pl.ds(15 * DEN_ROWS, DEN_ROWS_LAST)])

    return k1


def _make_k2(W, h_from_core, H):
    mesh = plsc.VectorSubcoreMesh(core_axis_name="c", subcore_axis_name="s",
                                  num_cores=NC, num_subcores=NS)
    out_type = jax.ShapeDtypeStruct((2, N, W), jnp.float32)
    scratch = [
        pltpu.VMEM((K2_CG, 3, G), jnp.int32),
        pltpu.VMEM((K2_CG, G), jnp.int32),
        pltpu.VMEM((K2_CE, W), jnp.float32),
        pltpu.VMEM((K2_CE,), jnp.float32),
        pltpu.VMEM_SHARED((N, W), jnp.float32),
        pltpu.SemaphoreType.DMA,
        pltpu.SemaphoreType.DMA,
    ]
    NV = W // 16

    @functools.partial(pl.kernel, out_type=out_type, mesh=mesh,
                       compiler_params=pltpu.CompilerParams(
                           use_tc_tiling_on_sc=False, needs_layout_passes=False),
                       scratch_types=scratch)
    def k2(xr_hbm, eG, ex_hbm, agg_out,
           eb, rows, msg, exbuf, agg_sp, sem0, sem1):
        c = lax.axis_index("c")
        s = lax.axis_index("s")
        c_off = c * (R * N)

        def zero_m16(t, _):
            r = t // NV
            v = (t % NV) * 16
            msg[r, pl.ds(v, 16)] = jnp.zeros((16,), jnp.float32)
            return _
        lax.fori_loop(0, K2_CE * NV, zero_m16, 0)

        base_row = s * DEN_ROWS

        nfull, rem = divmod(DEN_ROWS, K2_CE)
        nfull_l, rem_l = divmod(DEN_ROWS_LAST, K2_CE)

        @pl.when(s < 15)
        def _():
            for q in range(nfull):
                pltpu.sync_copy(msg, agg_sp.at[pl.ds(base_row + q * K2_CE, K2_CE)])
            if rem:
                pltpu.sync_copy(msg.at[pl.ds(0, rem)],
                                agg_sp.at[pl.ds(base_row + nfull * K2_CE, rem)])

        @pl.when(s == 15)
        def _():
            for q in range(nfull_l):
                pltpu.sync_copy(msg, agg_sp.at[pl.ds(base_row + q * K2_CE, K2_CE)])
            if rem_l:
                pltpu.sync_copy(msg.at[pl.ds(0, rem_l)],
                                agg_sp.at[pl.ds(base_row + nfull_l * K2_CE, rem_l)])

        plsc.subcore_barrier()

        def chunk(ci, _):
            base_g = s * K2_TG + ci * K2_CG
            pltpu.sync_copy(eG.at[pl.ds(base_g, K2_CG)], eb)

            def rows_body(t, _2):
                g = t // 8
                k = (t % 8) * 16
                sv = eb[g, 0, pl.ds(k, 16)]
                ev = eb[g, 2, pl.ds(k, 16)]
                rows[g, pl.ds(k, 16)] = c_off + ev * N + sv
                return _2
            lax.fori_loop(0, K2_CG * 8, rows_body, 0)

            edge0 = base_g * G
            if h_from_core:
                excp = pltpu.async_copy(ex_hbm.at[c, pl.ds(edge0, K2_CE)],
                                        exbuf, sem1)
            else:
                excp = pltpu.async_copy(ex_hbm.at[0, pl.ds(edge0, K2_CE)],
                                        exbuf, sem1)

            cps = []
            for j in range(K2_CG):
                cps.append(pltpu.async_copy(
                    xr_hbm.at[rows.at[j]], msg.at[pl.ds(j * G, G)], sem0))
            excp.wait()
            for cp in cps:
                cp.wait()

            def sc_body(t, _2):
                for u in range(4):
                    e = t * 4 + u
                    sv = plsc.load_gather(exbuf, [jnp.full((16,), e, jnp.int32)])
                    for v in range(NV):
                        blk = msg[e, pl.ds(v * 16, 16)]
                        msg[e, pl.ds(v * 16, 16)] = blk * sv
                return _2
            lax.fori_loop(0, K2_CE // 4, sc_body, 0)

            cps = []
            for j in range(K2_CG):
                cps.append(pltpu.async_copy(
                    msg.at[pl.ds(j * G, G)], agg_sp.at[eb.at[j, 1]], sem0,
                    add=True))
            for cp in cps:
                cp.wait()
            return _
        lax.fori_loop(0, K2_NCH, chunk, 0)

        plsc.subcore_barrier()

        @pl.when(s < 15)
        def _():
            pltpu.sync_copy(agg_sp.at[pl.ds(base_row, DEN_ROWS)],
                            agg_out.at[c, pl.ds(base_row, DEN_ROWS)])

        @pl.when(s == 15)
        def _():
            pltpu.sync_copy(agg_sp.at[pl.ds(base_row, DEN_ROWS_LAST)],
                            agg_out.at[c, pl.ds(base_row, DEN_ROWS_LAST)])

    return k2


KM_CG = 5
KM_CE = KM_CG * G
KM_NCH = K2_TG // KM_CG


def _make_km2():
    W = 16
    mesh = plsc.VectorSubcoreMesh(core_axis_name="c", subcore_axis_name="s",
                                  num_cores=NC, num_subcores=NS)
    out_type = (
        jax.ShapeDtypeStruct((2, N, W), jnp.float32),
        jax.ShapeDtypeStruct((2, N, 8), jnp.float32),
    )
    scratch = [
        pltpu.VMEM((KM_CG, 3, G), jnp.int32),
        pltpu.VMEM((KM_CG, G), jnp.int32),
        pltpu.VMEM((KM_CG, G), jnp.int32),
        pltpu.VMEM((KM_CG, G), jnp.int32),
        pltpu.VMEM((KM_CE, 16), jnp.float32),
        pltpu.VMEM((KM_CE, 16), jnp.float32),
        pltpu.VMEM((KM_CE, W), jnp.float32),
        pltpu.VMEM((KM_CE, 8), jnp.float32),
        pltpu.VMEM((KM_CE,), jnp.float32),
        pltpu.VMEM_SHARED((N, W), jnp.float32),
        pltpu.VMEM_SHARED((N, 8), jnp.float32),
        pltpu.SemaphoreType.DMA,
        pltpu.SemaphoreType.DMA,
    ]

    @functools.partial(pl.kernel, out_type=out_type, mesh=mesh,
                       compiler_params=pltpu.CompilerParams(
                           use_tc_tiling_on_sc=False, needs_layout_passes=False),
                       scratch_types=scratch)
    def km2(xr_hbm, qk_hbm, eG, agg_out, den_out,
            eb, rowsD, rowsS, rowsM, qkd, qks, msg, pair,
            exbuf, agg_sp, den_sp, sem0, sem1):
        c = lax.axis_index("c")
        s = lax.axis_index("s")
        c_off = c * (R * N)

        def zero_m16(t, _):
            msg[t, pl.ds(0, 16)] = jnp.zeros((16,), jnp.float32)
            return _
        lax.fori_loop(0, KM_CE, zero_m16, 0)
        _zero_wk(pair, KM_CE, 8)

        base_row = s * DEN_ROWS

        @pl.when(s < 15)
        def _():
            for q in range(DEN_ROWS // KM_CE):
                pltpu.sync_copy(msg, agg_sp.at[pl.ds(base_row + q * KM_CE, KM_CE)])
                pltpu.sync_copy(pair, den_sp.at[pl.ds(base_row + q * KM_CE, KM_CE)])
            rem = DEN_ROWS % KM_CE
            if rem:
                off = base_row + (DEN_ROWS // KM_CE) * KM_CE
                pltpu.sync_copy(msg.at[pl.ds(0, rem)], agg_sp.at[pl.ds(off, rem)])
                pltpu.sync_copy(pair.at[pl.ds(0, rem)], den_sp.at[pl.ds(off, rem)])

        @pl.when(s == 15)
        def _():
            for q in range(DEN_ROWS_LAST // KM_CE):
                pltpu.sync_copy(msg, agg_sp.at[pl.ds(base_row + q * KM_CE, KM_CE)])
                pltpu.sync_copy(pair, den_sp.at[pl.ds(base_row + q * KM_CE, KM_CE)])
            rem = DEN_ROWS_LAST % KM_CE
            if rem:
                off = base_row + (DEN_ROWS_LAST // KM_CE) * KM_CE
                pltpu.sync_copy(msg.at[pl.ds(0, rem)], agg_sp.at[pl.ds(off, rem)])
                pltpu.sync_copy(pair.at[pl.ds(0, rem)], den_sp.at[pl.ds(off, rem)])

        plsc.subcore_barrier()

        def chunk(ci, _):
            base_g = s * K2_TG + ci * KM_CG
            pltpu.sync_copy(eG.at[pl.ds(base_g, KM_CG)], eb)

            def rows_body(t, _2):
                g = t // 8
                k = (t % 8) * 16
                sv = eb[g, 0, pl.ds(k, 16)]
                dv = eb[g, 1, pl.ds(k, 16)]
                ev = eb[g, 2, pl.ds(k, 16)]
                evN = ev * N
                rowsD[g, pl.ds(k, 16)] = evN + dv
                rs = evN + sv
                rowsS[g, pl.ds(k, 16)] = rs
                rowsM[g, pl.ds(k, 16)] = rs + c_off
                return _2
            lax.fori_loop(0, KM_CG * 8, rows_body, 0)

            cps = []
            for j in range(KM_CG):
                cps.append(pltpu.async_copy(
                    qk_hbm.at[rowsD.at[j]], qkd.at[pl.ds(j * G, G)], sem0))
                cps.append(pltpu.async_copy(
                    qk_hbm.at[rowsS.at[j]], qks.at[pl.ds(j * G, G)], sem0))
                cps.append(pltpu.async_copy(
                    xr_hbm.at[rowsM.at[j]], msg.at[pl.ds(j * G, G)], sem1))
            for cp in cps:
                cp.wait()

            edge0 = base_g * G

            def ex_body(t, _2):
                rvec = t * 16 + _iota16()
                eid = edge0 + rvec
                valid = eid < E
                q = plsc.load_gather(qkd, [rvec, jnp.full((16,), 0, jnp.int32)])
                k = plsc.load_gather(qks, [rvec, jnp.full((16,), 1, jnp.int32)])
                sc = q + k
                a = jnp.where(sc >= 0.0, sc, 0.2 * sc)
                exv = jnp.where(valid, jnp.exp(a), 0.0)
                exbuf[pl.ds(t * 16, 16)] = exv
                plsc.store_scatter(pair, [rvec, jnp.full((16,), 0, jnp.int32)], exv)
                return _2
            lax.fori_loop(0, KM_CE // 16, ex_body, 0)

            def sc_body(t, _2):
                for u in range(4):
                    e = t * 4 + u
                    sv = plsc.load_gather(exbuf, [jnp.full((16,), e, jnp.int32)])
                    msg[e, pl.ds(0, 16)] = msg[e, pl.ds(0, 16)] * sv
                return _2
            lax.fori_loop(0, KM_CE // 4, sc_body, 0)

            cps = []
            for j in range(KM_CG):
                cps.append(pltpu.async_copy(
                    msg.at[pl.ds(j * G, G)], agg_sp.at[eb.at[j, 1]], sem0,
                    add=True))
                cps.append(pltpu.async_copy(
                    pair.at[pl.ds(j * G, G)], den_sp.at[eb.at[j, 1]], sem1,
                    add=True))
            for cp in cps:
                cp.wait()
            return _
        lax.fori_loop(0, KM_NCH, chunk, 0)

        plsc.subcore_barrier()

        @pl.when(s < 15)
        def _():
            pltpu.sync_copy(agg_sp.at[pl.ds(base_row, DEN_ROWS)],
                            agg_out.at[c, pl.ds(base_row, DEN_ROWS)])
            pltpu.sync_copy(den_sp.at[pl.ds(base_row, DEN_ROWS)],
                            den_out.at[c, pl.ds(base_row, DEN_ROWS)])

        @pl.when(s == 15)
        def _():
            pltpu.sync_copy(agg_sp.at[pl.ds(base_row, DEN_ROWS_LAST)],
                            agg_out.at[c, pl.ds(base_row, DEN_ROWS_LAST)])
            pltpu.sync_copy(den_sp.at[pl.ds(base_row, DEN_ROWS_LAST)],
                            den_out.at[c, pl.ds(base_row, DEN_ROWS_LAST)])

    return km2


def _tc_proj(xs, Ws, bs):
    NB = 1000

    def body(x_ref, w_ref, b_ref, o_ref):
        m = jnp.dot(x_ref[0], w_ref[0], preferred_element_type=jnp.float32)
        o_ref[0] = jnp.maximum(m + b_ref[0, 0], 0.0)

    return pl.pallas_call(
        body,
        grid=(2, N0 // NB),
        in_specs=[
            pl.BlockSpec((1, NB, 256), lambda t, nb: (t, nb, 0)),
            pl.BlockSpec((1, 256, 64), lambda t, nb: (t, 0, 0)),
            pl.BlockSpec((1, 1, 64), lambda t, nb: (t, 0, 0)),
        ],
        out_specs=pl.BlockSpec((1, NB, 64), lambda t, nb: (t, nb, 0)),
        out_shape=jax.ShapeDtypeStruct((2, N0, 64), jnp.float32),
    )(xs, Ws, bs)


def _tc_tables(x, w, qkw, W):
    NB = 2000
    F = x.shape[1]

    def body(x_ref, w_ref, qkw_ref, xr_ref, qk_ref):
        m = jnp.dot(x_ref[...], w_ref[0], preferred_element_type=jnp.float32)
        xr_ref[0, 0] = m[:, :W]
        xr_ref[1, 0] = m[:, W:]
        qk_ref[0] = jnp.dot(x_ref[...], qkw_ref[0],
                            preferred_element_type=jnp.float32)

    return pl.pallas_call(
        body,
        grid=(R, N // NB),
        in_specs=[
            pl.BlockSpec((NB, F), lambda r, nb: (nb, 0)),
            pl.BlockSpec((1, F, 2 * W), lambda r, nb: (r, 0, 0)),
            pl.BlockSpec((1, F, 16), lambda r, nb: (r, 0, 0)),
        ],
        out_specs=[
            pl.BlockSpec((2, 1, NB, W), lambda r, nb: (0, r, nb, 0)),
            pl.BlockSpec((1, NB, 16), lambda r, nb: (r, nb, 0)),
        ],
        out_shape=[
            jax.ShapeDtypeStruct((2, R, N, W), jnp.float32),
            jax.ShapeDtypeStruct((R, N, 16), jnp.float32),
        ],
    )(x, w, qkw)


def _tc_layer1_epilogue(agg, den, b1):
    NB = 2000

    def body(a_ref, d_ref, b_ref, o_ref):
        dsum = d_ref[0] + d_ref[1]
        r0 = 1.0 / (dsum[:, 0:1] + 1e-16)
        r1 = 1.0 / (dsum[:, 1:2] + 1e-16)
        y = jnp.concatenate([a_ref[0] * r0, a_ref[1] * r1], axis=1)
        o_ref[...] = jnp.maximum(y + b_ref[...], 0.0)

    return pl.pallas_call(
        body,
        grid=(N // NB,),
        in_specs=[
            pl.BlockSpec((2, NB, 32), lambda nb: (0, nb, 0)),
            pl.BlockSpec((2, NB, 8), lambda nb: (0, nb, 0)),
            pl.BlockSpec((64,), lambda nb: (0,)),
        ],
        out_specs=pl.BlockSpec((NB, 64), lambda nb: (nb, 0)),
        out_shape=jax.ShapeDtypeStruct((N, 64), jnp.float32),
    )(agg, den, b1)


def _tc_head(agg2, den2, b2, lin_W, lin_b):
    NB = 1000

    def body(a_ref, d_ref, b_ref, lw_ref, lb_ref, o_ref):
        r0 = 1.0 / (d_ref[0][:, 0:1] + 1e-16)
        y = jnp.concatenate([a_ref[0], a_ref[1]], axis=1) * r0
        y = jnp.maximum(y + b_ref[...], 0.0)
        o_ref[...] = jnp.dot(y, lw_ref[...],
                             preferred_element_type=jnp.float32) + lb_ref[...]

    return pl.pallas_call(
        body,
        grid=(N0 // NB,),
        in_specs=[
            pl.BlockSpec((2, NB, 16), lambda nb: (0, nb, 0)),
            pl.BlockSpec((2, NB, 8), lambda nb: (0, nb, 0)),
            pl.BlockSpec((32,), lambda nb: (0,)),
            pl.BlockSpec((32, 8), lambda nb: (0, 0)),
            pl.BlockSpec((8,), lambda nb: (0,)),
        ],
        out_specs=pl.BlockSpec((NB, 8), lambda nb: (nb, 0)),
        out_shape=jax.ShapeDtypeStruct((N0, 8), jnp.float32),
    )(agg2, den2, b2, lin_W, lin_b)


_SC_KERNELS = {}


def _sc_kernels():
    if not _SC_KERNELS:
        _SC_KERNELS["k1l1"] = _make_k1(2)
        _SC_KERNELS["k2l1"] = _make_k2(32, True, 2)
        _SC_KERNELS["km2"] = _make_km2()
    return _SC_KERNELS


def _qkw(w, q, k, H):
    qk = jnp.einsum('rio,oh->rih', w, jnp.concatenate([q, k], axis=1))
    pad = 16 - 2 * H
    return jnp.pad(qk, ((0, 0), (0, 0), (0, pad)))


def kernel(x0, x1, edge_index, edge_type, proj0_W, proj0_b, proj1_W, proj1_b,
           w1, q1, k1, b1, w2, q2, k2, b2, lin_W, lin_b):
    ei = edge_index.astype(jnp.int32)
    et = edge_type.astype(jnp.int32)
    padn = E_PAD - E
    eG = jnp.stack([
        jnp.pad(ei[0], (0, padn)).reshape(NGROUPS, G),
        jnp.pad(ei[1], (0, padn)).reshape(NGROUPS, G),
        jnp.pad(et, (0, padn)).reshape(NGROUPS, G),
    ], axis=1)

    xs = jnp.stack([x0, x1])
    Ws = jnp.stack([proj0_W, proj1_W])
    bs = jnp.stack([proj0_b, proj1_b]).reshape(2, 1, 64)

    x = _tc_proj(xs, Ws, bs).reshape(N, 64)

    xr1, qk1 = _tc_tables(x, w1, _qkw(w1, q1, k1, 2), 32)
    xr1 = xr1.reshape(2 * R * N, 32)
    qk1 = qk1.reshape(R * N, 16)
    sck = _sc_kernels()
    ex1, den1 = sck["k1l1"](qk1, eG)
    agg1 = sck["k2l1"](xr1, eG, ex1)
    x2 = _tc_layer1_epilogue(agg1, den1, b1)

    xr2, qk2 = _tc_tables(x2, w2, _qkw(w2, q2, k2, 1), 16)
    xr2 = xr2.reshape(2 * R * N, 16)
    qk2 = qk2.reshape(R * N, 16)
    agg2, den2 = sck["km2"](xr2, qk2, eG)

    return _tc_head(agg2, den2, b2, lin_W, lin_b)
